# Initial kernel scaffold; baseline (speedup 1.0000x reference)
import jax
import jax.numpy as jnp
from jax.experimental import pallas as pl


def kernel(node_feats, node_attrs_total, node_attrs_slice, edge_feats, edge_attrs, edge_index, cutoff, W_up, Wr0, Wr1, Wr2, W_lin, W_sc):
    raise NotImplementedError("write your pallas kernel here")


# speedup vs baseline: 5.7299x; 5.7299x over previous
import functools

import jax
import jax.numpy as jnp
from jax import lax
from jax.experimental import pallas as pl
from jax.experimental.pallas import tpu as pltpu
from jax.experimental.pallas import tpu_sc as plsc

N = 10000
E = 160000
D = 128
S = 4
NRB = 8
H = 64
NE = 4
DSC = 128
AVG_INV = 1.0 / 16.0

NC, NS = 2, 16
NW = NC * NS
EPW = E // NW
CH = 128
NFULL = EPW // CH
TAIL = EPW - NFULL * CH
NP = 10240
RZ = NP // NS

@functools.cache
def _sc_mesh():
  return plsc.VectorSubcoreMesh(
      core_axis_name="c", subcore_axis_name="s", num_cores=NC, num_subcores=NS)


def _mm_body(a_ref, b_ref, o_ref):
  o_ref[...] = jnp.dot(a_ref[...], b_ref[...], preferred_element_type=jnp.float32)


def _linear_up(node_feats, W_up):
  return pl.pallas_call(
      _mm_body,
      out_shape=jax.ShapeDtypeStruct((N, D), jnp.float32),
  )(node_feats, W_up)


def _final_body(p_ref, natot_ref, wsc_ref, mi_ref, scs_ref):
  m = p_ref[0] + p_ref[1]
  mi_ref[...] = m
  acc = jnp.zeros((m.shape[0], DSC), jnp.float32)
  for q in range(NE):
    acc += jnp.dot(m, wsc_ref[q], preferred_element_type=jnp.float32) \
        * natot_ref[:, q:q + 1]
  scs_ref[...] = acc


def _final_stage(partials, natot, W_sc):
  BN = 2000
  grid = (N // BN,)
  return pl.pallas_call(
      _final_body,
      grid=grid,
      in_specs=[
          pl.BlockSpec((NC, BN, D), lambda i: (0, i, 0)),
          pl.BlockSpec((BN, NE), lambda i: (i, 0)),
          pl.BlockSpec((NE, D, DSC), lambda i: (0, 0, 0)),
      ],
      out_specs=[
          pl.BlockSpec((BN, D), lambda i: (i, 0)),
          pl.BlockSpec((BN, DSC), lambda i: (i, 0)),
      ],
      out_shape=[
          jax.ShapeDtypeStruct((N, D), jnp.float32),
          jax.ShapeDtypeStruct((N, DSC), jnp.float32),
      ],
  )(partials, natot, W_sc)


def _edge_body(ef_ref, attr_ref, cut_ref, xe_ref, wr0_ref, wr1_ref, wr2_ref,
               wlin_ref, v_ref):
  h = jax.nn.silu(jnp.dot(ef_ref[...], wr0_ref[...],
                          preferred_element_type=jnp.float32))
  h = jax.nn.silu(jnp.dot(h, wr1_ref[...], preferred_element_type=jnp.float32))
  cw = jnp.dot(h, wr2_ref[...], preferred_element_type=jnp.float32) \
      * cut_ref[...]
  xe = xe_ref[...]
  attr = attr_ref[...]
  acc = jnp.zeros((xe.shape[0], D), jnp.float32)
  for s in range(S):
    t = xe * attr[:, s:s + 1] * cw[:, s * D:(s + 1) * D]
    acc += jnp.dot(t, wlin_ref[s], preferred_element_type=jnp.float32)
  v_ref[...] = acc * AVG_INV


def _edge_stage(edge_feats, edge_attrs, cutoff, xe, Wr0, Wr1, Wr2p, W_linp):
  BE = 2000
  grid = (E // BE,)
  return pl.pallas_call(
      _edge_body,
      grid=grid,
      in_specs=[
          pl.BlockSpec((BE, NRB), lambda i: (i, 0)),
          pl.BlockSpec((BE, S), lambda i: (i, 0)),
          pl.BlockSpec((BE, 1), lambda i: (i, 0)),
          pl.BlockSpec((BE, D), lambda i: (i, 0)),
          pl.BlockSpec((NRB, H), lambda i: (0, 0)),
          pl.BlockSpec((H, H), lambda i: (0, 0)),
          pl.BlockSpec((H, S * D), lambda i: (0, 0)),
          pl.BlockSpec((S, D, D), lambda i: (0, 0, 0)),
      ],
      out_specs=pl.BlockSpec((BE, D), lambda i: (i, 0)),
      out_shape=jax.ShapeDtypeStruct((E, D), jnp.float32),
  )(edge_feats, edge_attrs, cutoff, xe, Wr0, Wr1, Wr2p, W_linp)


@functools.cache
def _build_gather():
  return functools.partial(
      pl.kernel,
      out_type=jax.ShapeDtypeStruct((E, D), jnp.float32),
      mesh=_sc_mesh(),
      scratch_types=[
          pltpu.VMEM((CH,), jnp.int32),
          pltpu.VMEM((CH, D), jnp.float32),
          pltpu.VMEM((TAIL,), jnp.int32),
          pltpu.VMEM((TAIL, D), jnp.float32),
          pltpu.SemaphoreType.DMA,
      ],
  )(_gather_body)


def _gather_body(x_hbm, src_hbm, xe_hbm, idx_v, rows_v, idx_t, rows_t, sem):
  wid = lax.axis_index("s") * NC + lax.axis_index("c")
  base = wid * EPW

  def chunk(c, carry):
    off = base + c * CH
    pltpu.sync_copy(src_hbm.at[pl.ds(off, CH)], idx_v)
    pltpu.async_copy(x_hbm.at[idx_v], rows_v, sem).wait()
    pltpu.sync_copy(rows_v, xe_hbm.at[pl.ds(off, CH)])
    return carry

  lax.fori_loop(0, NFULL, chunk, 0)
  off = base + NFULL * CH
  pltpu.sync_copy(src_hbm.at[pl.ds(off, TAIL)], idx_t)
  pltpu.async_copy(x_hbm.at[idx_t], rows_t, sem).wait()
  pltpu.sync_copy(rows_t, xe_hbm.at[pl.ds(off, TAIL)])


@functools.cache
def _build_scatter():
  return functools.partial(
      pl.kernel,
      out_type=jax.ShapeDtypeStruct((NC, NP, D), jnp.float32),
      mesh=_sc_mesh(),
      scratch_types=[
          pltpu.VMEM((CH,), jnp.int32),
          pltpu.VMEM((CH, D), jnp.float32),
          pltpu.VMEM((TAIL,), jnp.int32),
          pltpu.VMEM((TAIL, D), jnp.float32),
          pltpu.VMEM_SHARED((NP, D), jnp.float32),
          pltpu.SemaphoreType.DMA,
      ],
  )(_scatter_body)


def _scatter_body(v_hbm, dst_hbm, zeros_hbm, out_hbm, idx_v, rows_v, idx_t,
                  rows_t, acc_sh, sem):
  cid = lax.axis_index("c")
  sid = lax.axis_index("s")
  pltpu.sync_copy(zeros_hbm.at[pl.ds(sid * RZ, RZ)],
                  acc_sh.at[pl.ds(sid * RZ, RZ)])
  plsc.subcore_barrier()

  wid = sid * NC + cid
  base = wid * EPW

  def chunk(c, carry):
    off = base + c * CH
    pltpu.sync_copy(dst_hbm.at[pl.ds(off, CH)], idx_v)
    pltpu.sync_copy(v_hbm.at[pl.ds(off, CH)], rows_v)
    pltpu.sync_copy(rows_v, acc_sh.at[idx_v], add=True)
    return carry

  lax.fori_loop(0, NFULL, chunk, 0)
  off = base + NFULL * CH
  pltpu.sync_copy(dst_hbm.at[pl.ds(off, TAIL)], idx_t)
  pltpu.sync_copy(v_hbm.at[pl.ds(off, TAIL)], rows_t)
  pltpu.sync_copy(rows_t, acc_sh.at[idx_t], add=True)
  plsc.subcore_barrier()
  pltpu.sync_copy(acc_sh.at[pl.ds(sid * RZ, RZ)],
                  out_hbm.at[cid, pl.ds(sid * RZ, RZ)])


def kernel(node_feats, node_attrs_total, node_attrs_slice, edge_feats,
           edge_attrs, edge_index, cutoff, W_up, Wr0, Wr1, Wr2, W_lin, W_sc):
  del node_attrs_slice
  src = edge_index[0]
  dst = edge_index[1]
  Wr2p = Wr2.reshape(H, D, S).transpose(0, 2, 1).reshape(H, S * D)
  W_linp = W_lin.reshape(D, S, D).transpose(1, 0, 2)

  x = _linear_up(node_feats, W_up)
  xe = _build_gather()(x, src)
  v = _edge_stage(edge_feats, edge_attrs, cutoff, xe, Wr0, Wr1, Wr2p, W_linp)
  zeros = jnp.zeros((NP, D), jnp.float32)
  partials = _build_scatter()(v, dst, zeros)
  m_i, scs = _final_stage(partials, node_attrs_total, W_sc)
  return (m_i, scs)


# speedup vs baseline: 6.1398x; 1.0715x over previous
import functools

import jax
import jax.numpy as jnp
from jax import lax
from jax.experimental import pallas as pl
from jax.experimental.pallas import tpu as pltpu
from jax.experimental.pallas import tpu_sc as plsc

N = 10000
E = 160000
D = 128
S = 4
NRB = 8
H = 64
NE = 4
DSC = 128
AVG_INV = 1.0 / 16.0

NC, NS = 2, 16
NW = NC * NS
EPW = E // NW
CH = 128
NFULL = EPW // CH
TAIL = EPW - NFULL * CH
NP = 10240
RZ = NP // NS

@functools.cache
def _sc_mesh():
  return plsc.VectorSubcoreMesh(
      core_axis_name="c", subcore_axis_name="s", num_cores=NC, num_subcores=NS)


def _mm_body(a_ref, b_ref, o_ref):
  o_ref[...] = jnp.dot(a_ref[...], b_ref[...], preferred_element_type=jnp.float32)


def _linear_up(node_feats, W_up):
  return pl.pallas_call(
      _mm_body,
      out_shape=jax.ShapeDtypeStruct((N, D), jnp.float32),
  )(node_feats, W_up)


def _final_body(p_ref, natot_ref, wsc_ref, mi_ref, scs_ref):
  m = p_ref[0] + p_ref[1]
  mi_ref[...] = m
  acc = jnp.zeros((m.shape[0], DSC), jnp.float32)
  for q in range(NE):
    acc += jnp.dot(m, wsc_ref[q], preferred_element_type=jnp.float32) \
        * natot_ref[:, q:q + 1]
  scs_ref[...] = acc


def _final_stage(partials, natot, W_sc):
  BN = 2000
  grid = (N // BN,)
  return pl.pallas_call(
      _final_body,
      grid=grid,
      in_specs=[
          pl.BlockSpec((NC, BN, D), lambda i: (0, i, 0)),
          pl.BlockSpec((BN, NE), lambda i: (i, 0)),
          pl.BlockSpec((NE, D, DSC), lambda i: (0, 0, 0)),
      ],
      out_specs=[
          pl.BlockSpec((BN, D), lambda i: (i, 0)),
          pl.BlockSpec((BN, DSC), lambda i: (i, 0)),
      ],
      out_shape=[
          jax.ShapeDtypeStruct((N, D), jnp.float32),
          jax.ShapeDtypeStruct((N, DSC), jnp.float32),
      ],
  )(partials, natot, W_sc)


def _edge_body(ef_ref, attr_ref, cut_ref, xe_ref, wr0_ref, wr1_ref, wr2_ref,
               wlin_ref, v_ref):
  h = jax.nn.silu(jnp.dot(ef_ref[...], wr0_ref[...],
                          preferred_element_type=jnp.float32))
  h = jax.nn.silu(jnp.dot(h, wr1_ref[...], preferred_element_type=jnp.float32))
  cw = jnp.dot(h, wr2_ref[...], preferred_element_type=jnp.float32) \
      * cut_ref[...]
  xe = xe_ref[...]
  attr = attr_ref[...]
  acc = jnp.zeros((xe.shape[0], D), jnp.float32)
  for s in range(S):
    t = xe * attr[:, s:s + 1] * cw[:, s * D:(s + 1) * D]
    acc += jnp.dot(t, wlin_ref[s], preferred_element_type=jnp.float32)
  v_ref[...] = acc * AVG_INV


def _edge_stage(edge_feats, edge_attrs, cutoff, xe, Wr0, Wr1, Wr2p, W_linp):
  BE = 2000
  grid = (E // BE,)
  return pl.pallas_call(
      _edge_body,
      grid=grid,
      in_specs=[
          pl.BlockSpec((BE, NRB), lambda i: (i, 0)),
          pl.BlockSpec((BE, S), lambda i: (i, 0)),
          pl.BlockSpec((BE, 1), lambda i: (i, 0)),
          pl.BlockSpec((BE, D), lambda i: (i, 0)),
          pl.BlockSpec((NRB, H), lambda i: (0, 0)),
          pl.BlockSpec((H, H), lambda i: (0, 0)),
          pl.BlockSpec((H, S * D), lambda i: (0, 0)),
          pl.BlockSpec((S, D, D), lambda i: (0, 0, 0)),
      ],
      out_specs=pl.BlockSpec((BE, D), lambda i: (i, 0)),
      out_shape=jax.ShapeDtypeStruct((E, D), jnp.float32),
  )(edge_feats, edge_attrs, cutoff, xe, Wr0, Wr1, Wr2p, W_linp)


@functools.cache
def _build_gather():
  return functools.partial(
      pl.kernel,
      out_type=jax.ShapeDtypeStruct((E, D), jnp.float32),
      mesh=_sc_mesh(),
      scratch_types=[
          pltpu.VMEM((CH,), jnp.int32),
          pltpu.VMEM((CH,), jnp.int32),
          pltpu.VMEM((CH, D), jnp.float32),
          pltpu.VMEM((CH, D), jnp.float32),
          pltpu.VMEM((TAIL,), jnp.int32),
          pltpu.VMEM((TAIL, D), jnp.float32),
          pltpu.SemaphoreType.DMA,
          pltpu.SemaphoreType.DMA,
          pltpu.SemaphoreType.DMA,
      ],
  )(_gather_body)


def _gather_body(x_hbm, src_hbm, xe_hbm, idx0, idx1, rows0, rows1, idx_t,
                 rows_t, sem0, sem1, sem_t):
  wid = lax.axis_index("s") * NC + lax.axis_index("c")
  base = wid * EPW
  idx = (idx0, idx1)
  rows = (rows0, rows1)
  sems = (sem0, sem1)

  def fire(p, c):
    off = base + c * CH
    pltpu.sync_copy(src_hbm.at[pl.ds(off, CH)], idx[p])
    pltpu.async_copy(x_hbm.at[idx[p]], rows[p], sems[p])

  def drain(p, c):
    off = base + c * CH
    pltpu.make_async_copy(x_hbm.at[idx[p]], rows[p], sems[p]).wait()
    pltpu.sync_copy(rows[p], xe_hbm.at[pl.ds(off, CH)])

  for p in range(2):
    fire(p, p)

  def pair(t, carry):
    c = 2 * t
    for p in range(2):
      drain(p, c + p)
      fire(p, c + 2 + p)
    return carry

  lax.fori_loop(0, (NFULL - 3) // 2, pair, 0)
  drain(0, NFULL - 3)
  fire(0, NFULL - 1)
  drain(1, NFULL - 2)
  off = base + NFULL * CH
  pltpu.sync_copy(src_hbm.at[pl.ds(off, TAIL)], idx_t)
  pltpu.async_copy(x_hbm.at[idx_t], rows_t, sem_t)
  drain(0, NFULL - 1)
  pltpu.make_async_copy(x_hbm.at[idx_t], rows_t, sem_t).wait()
  pltpu.sync_copy(rows_t, xe_hbm.at[pl.ds(off, TAIL)])


@functools.cache
def _build_scatter():
  return functools.partial(
      pl.kernel,
      out_type=jax.ShapeDtypeStruct((NC, NP, D), jnp.float32),
      mesh=_sc_mesh(),
      scratch_types=[
          pltpu.VMEM((CH,), jnp.int32),
          pltpu.VMEM((CH,), jnp.int32),
          pltpu.VMEM((CH, D), jnp.float32),
          pltpu.VMEM((CH, D), jnp.float32),
          pltpu.VMEM((TAIL,), jnp.int32),
          pltpu.VMEM((TAIL, D), jnp.float32),
          pltpu.VMEM_SHARED((NP, D), jnp.float32),
          pltpu.SemaphoreType.DMA,
          pltpu.SemaphoreType.DMA,
          pltpu.SemaphoreType.DMA,
      ],
  )(_scatter_body)


def _scatter_body(v_hbm, dst_hbm, zeros_hbm, out_hbm, idx0, idx1, rows0,
                  rows1, idx_t, rows_t, acc_sh, sem0, sem1, sem_t):
  cid = lax.axis_index("c")
  sid = lax.axis_index("s")
  pltpu.sync_copy(zeros_hbm.at[pl.ds(sid * RZ, RZ)],
                  acc_sh.at[pl.ds(sid * RZ, RZ)])
  plsc.subcore_barrier()

  wid = sid * NC + cid
  base = wid * EPW
  idx = (idx0, idx1)
  rows = (rows0, rows1)
  sems = (sem0, sem1)

  def fire(p, c):
    off = base + c * CH
    pltpu.sync_copy(dst_hbm.at[pl.ds(off, CH)], idx[p])
    pltpu.async_copy(v_hbm.at[pl.ds(off, CH)], rows[p], sems[p])

  def drain(p, c):
    off = base + c * CH
    pltpu.make_async_copy(v_hbm.at[pl.ds(off, CH)], rows[p], sems[p]).wait()
    pltpu.sync_copy(rows[p], acc_sh.at[idx[p]], add=True)

  for p in range(2):
    fire(p, p)

  def pair(t, carry):
    c = 2 * t
    for p in range(2):
      drain(p, c + p)
      fire(p, c + 2 + p)
    return carry

  lax.fori_loop(0, (NFULL - 3) // 2, pair, 0)
  drain(0, NFULL - 3)
  fire(0, NFULL - 1)
  drain(1, NFULL - 2)
  off = base + NFULL * CH
  pltpu.sync_copy(dst_hbm.at[pl.ds(off, TAIL)], idx_t)
  pltpu.async_copy(v_hbm.at[pl.ds(off, TAIL)], rows_t, sem_t)
  drain(0, NFULL - 1)
  pltpu.make_async_copy(v_hbm.at[pl.ds(off, TAIL)], rows_t, sem_t).wait()
  pltpu.sync_copy(rows_t, acc_sh.at[idx_t], add=True)
  plsc.subcore_barrier()
  pltpu.sync_copy(acc_sh.at[pl.ds(sid * RZ, RZ)],
                  out_hbm.at[cid, pl.ds(sid * RZ, RZ)])


def kernel(node_feats, node_attrs_total, node_attrs_slice, edge_feats,
           edge_attrs, edge_index, cutoff, W_up, Wr0, Wr1, Wr2, W_lin, W_sc):
  del node_attrs_slice
  src = edge_index[0]
  dst = edge_index[1]
  Wr2p = Wr2.reshape(H, D, S).transpose(0, 2, 1).reshape(H, S * D)
  W_linp = W_lin.reshape(D, S, D).transpose(1, 0, 2)

  x = _linear_up(node_feats, W_up)
  xe = _build_gather()(x, src)
  v = _edge_stage(edge_feats, edge_attrs, cutoff, xe, Wr0, Wr1, Wr2p, W_linp)
  zeros = jnp.zeros((NP, D), jnp.float32)
  partials = _build_scatter()(v, dst, zeros)
  m_i, scs = _final_stage(partials, node_attrs_total, W_sc)
  return (m_i, scs)
